# baseline (speedup 1.0000x reference)
import functools

import jax
import jax.numpy as jnp
from jax import lax
from jax.experimental import pallas as pl
from jax.experimental.pallas import tpu as pltpu
from jax.experimental.pallas import tpu_sc as plsc

N = 2048
D = 768
H = 3072
E = 64
T = 64
NT = 128
PAD = NT * T
CH = 512

NC, NS = 2, 16
NW = NC * NS
TPW = N // NW


def _router_body(xf_ref, rw_ref, rb_ref,
                 g0_ref, g1_ref, p0_ref, p1_ref, te_ref, tot_ref, tid_ref):
    xf = xf_ref[...]
    logits = jnp.dot(xf, rw_ref[...], preferred_element_type=jnp.float32)
    logits = logits + rb_ref[...]
    m = jnp.max(logits, axis=-1, keepdims=True)
    ex = jnp.exp(logits - m)
    p = ex / jnp.sum(ex, axis=-1, keepdims=True)

    lane = lax.broadcasted_iota(jnp.int32, (N, E), 1).astype(jnp.float32)
    m1 = jnp.max(p, axis=-1, keepdims=True)
    i1 = jnp.min(jnp.where(p == m1, lane, float(E + 1)), axis=-1, keepdims=True)
    pmask = jnp.where(lane == i1, -1.0, p)
    m2 = jnp.max(pmask, axis=-1, keepdims=True)
    i2 = jnp.min(jnp.where(pmask == m2, lane, float(E + 1)), axis=-1,
                 keepdims=True)
    g0_ref[...] = jnp.broadcast_to(m1, (N, 16))
    g1_ref[...] = jnp.broadcast_to(m2, (N, 16))
    tid_ref[...] = lax.broadcasted_iota(jnp.int32, (N, 128), 0)

    ea = jnp.concatenate([i1, i2], axis=0)
    lane_c = lax.broadcasted_iota(jnp.int32, (CH, E), 1).astype(jnp.float32)
    r = lax.broadcasted_iota(jnp.int32, (CH, CH), 0).astype(jnp.float32)
    c = lax.broadcasted_iota(jnp.int32, (CH, CH), 1).astype(jnp.float32)
    Ls = (r > c).astype(jnp.float32)

    carry = jnp.zeros((1, E), jnp.float32)
    ranks = []
    for k in range(2 * N // CH):
        Ok = (ea[k * CH:(k + 1) * CH] == lane_c).astype(jnp.float32)
        Ck = jnp.dot(Ls, Ok, preferred_element_type=jnp.float32) + carry
        carry = carry + jnp.sum(Ok, axis=0, keepdims=True)
        ranks.append(jnp.sum(Ck * Ok, axis=-1, keepdims=True))
    rank = jnp.concatenate(ranks, axis=0)
    counts = carry

    tpe = jnp.floor((counts + (T - 1)) * (1.0 / T))
    er = lax.broadcasted_iota(jnp.int32, (E, E), 0).astype(jnp.float32)
    ec = lax.broadcasted_iota(jnp.int32, (E, E), 1).astype(jnp.float32)
    Le = (er > ec).astype(jnp.float32)
    cum_excl = jnp.dot(Le, tpe.reshape(E, 1),
                       preferred_element_type=jnp.float32).reshape(1, E)
    cum_incl = cum_excl + tpe
    total = jnp.sum(tpe)
    padded_off = float(T) * cum_excl

    poa = jnp.zeros((0, 1), jnp.float32)
    poas = []
    for k in range(2 * N // CH):
        Ok = (ea[k * CH:(k + 1) * CH] == lane_c).astype(jnp.float32)
        poas.append(jnp.sum(Ok * padded_off, axis=-1, keepdims=True))
    poa = jnp.concatenate(poas, axis=0)
    pos = (poa + rank).astype(jnp.int32)
    p0_ref[...] = pos[:N]
    p1_ref[...] = pos[N:]

    ti = lax.broadcasted_iota(jnp.int32, (NT, E), 0).astype(jnp.float32)
    te = jnp.sum((cum_incl <= ti).astype(jnp.float32), axis=-1, keepdims=True)
    e_col = lax.broadcasted_iota(jnp.int32, (1, E), 1).astype(jnp.float32)
    last_e = jnp.max(jnp.where(tpe > 0, e_col, -1.0))
    te = jnp.minimum(te, last_e)
    te_ref[...] = te.astype(jnp.int32)
    tot_ref[...] = total.astype(jnp.int32).reshape(1, 1)


_router = pl.pallas_call(
    _router_body,
    out_shape=(
        jax.ShapeDtypeStruct((N, 16), jnp.float32),
        jax.ShapeDtypeStruct((N, 16), jnp.float32),
        jax.ShapeDtypeStruct((N, 1), jnp.int32),
        jax.ShapeDtypeStruct((N, 1), jnp.int32),
        jax.ShapeDtypeStruct((NT, 1), jnp.int32),
        jax.ShapeDtypeStruct((1, 1), jnp.int32),
        jax.ShapeDtypeStruct((N, 128), jnp.int32),
    ),
)


def _dispatch_body(tid_hbm, p0_hbm, p1_hbm, rid_hbm, rows_v, i0_v, i1_v,
                   s0, s1, sr):
    wid = lax.axis_index("s") * NC + lax.axis_index("c")
    base = wid * TPW
    cr = pltpu.async_copy(tid_hbm.at[pl.ds(base, TPW)], rows_v, sr)
    c0i = pltpu.async_copy(p0_hbm.at[pl.ds(base, TPW)], i0_v, s0)
    c1i = pltpu.async_copy(p1_hbm.at[pl.ds(base, TPW)], i1_v, s1)
    cr.wait()
    c0i.wait()
    c1i.wait()
    c0 = pltpu.async_copy(rows_v, rid_hbm.at[i0_v], s0)
    c1 = pltpu.async_copy(rows_v, rid_hbm.at[i1_v], s1)
    c0.wait()
    c1.wait()


@functools.cache
def _get_dispatch():
    mesh = plsc.VectorSubcoreMesh(core_axis_name="c", subcore_axis_name="s",
                                  num_cores=NC, num_subcores=NS)
    return pl.kernel(
        _dispatch_body,
        out_type=jax.ShapeDtypeStruct((PAD, 128), jnp.int32),
        mesh=mesh,
        scratch_types=[
            pltpu.VMEM((TPW, 128), jnp.int32),
            pltpu.VMEM((TPW,), jnp.int32),
            pltpu.VMEM((TPW,), jnp.int32),
            pltpu.SemaphoreType.DMA,
            pltpu.SemaphoreType.DMA,
            pltpu.SemaphoreType.DMA,
        ],
    )


_INV_SQRT2 = 0.7071067811865476


_H2 = H // 2


def _gmm_body(te_ref, tot_ref, rid_hbm, xf_ref, w1_hbm, b1_hbm, w2_hbm,
              b2_hbm, ys_hbm):
    total = tot_ref[0, 0]

    def inner(rid_blk, w1_blk, b1_blk, w2_blk, b2_blk, ys_blk):
        ids = rid_blk[...][:, :1]
        sel = (ids == lax.broadcasted_iota(jnp.int32, (T, N), 1))
        sel = sel.astype(jnp.float32)
        xt = jnp.dot(sel, xf_ref[...], preferred_element_type=jnp.float32)
        h = jnp.dot(xt, w1_blk[0],
                    preferred_element_type=jnp.float32) + b1_blk[0]
        h = 0.5 * h * (1.0 + lax.erf(h * _INV_SQRT2))
        yt = jnp.dot(h, w2_blk[0], preferred_element_type=jnp.float32)
        ys_blk[...] = yt + b2_blk[0]

    wbuf3 = pl.Buffered(buffer_count=3, use_lookahead=True)
    wbuf2 = pl.Buffered(buffer_count=2, use_lookahead=True)
    pipeline = pltpu.emit_pipeline(
        inner,
        grid=(total,),
        in_specs=[
            pl.BlockSpec((T, 128), lambda i: (i, 0)),
            pl.BlockSpec((1, D, H), lambda i: (te_ref[i, 0], 0, 0),
                         pipeline_mode=wbuf3),
            pl.BlockSpec((1, 1, H), lambda i: (te_ref[i, 0], 0, 0)),
            pl.BlockSpec((1, H, D), lambda i: (te_ref[i, 0], 0, 0),
                         pipeline_mode=wbuf2),
            pl.BlockSpec((1, 1, D), lambda i: (te_ref[i, 0], 0, 0)),
        ],
        out_specs=[pl.BlockSpec((T, D), lambda i: (i, 0))],
    )
    pipeline(rid_hbm, w1_hbm, b1_hbm, w2_hbm, b2_hbm, ys_hbm)


_gmm = pl.pallas_call(
    _gmm_body,
    in_specs=[
        pl.BlockSpec(memory_space=pltpu.SMEM),
        pl.BlockSpec(memory_space=pltpu.SMEM),
        pl.BlockSpec(memory_space=pl.ANY),
        pl.BlockSpec((N, D), lambda: (0, 0)),
        pl.BlockSpec(memory_space=pl.ANY),
        pl.BlockSpec(memory_space=pl.ANY),
        pl.BlockSpec(memory_space=pl.ANY),
        pl.BlockSpec(memory_space=pl.ANY),
    ],
    out_specs=pl.BlockSpec(memory_space=pl.ANY),
    out_shape=jax.ShapeDtypeStruct((PAD, D), jnp.float32),
)


_CHT = 32


def _combine_body(ys_hbm, p0_hbm, p1_hbm, g0_hbm, g1_hbm, out_hbm,
                  r0a_v, r1a_v, r0b_v, r1b_v,
                  i0a_v, i1a_v, i0b_v, i1b_v, g0_v, g1_v,
                  s0a, s1a, s0b, s1b):
    wid = lax.axis_index("s") * NC + lax.axis_index("c")
    base = wid * TPW

    pltpu.sync_copy(p0_hbm.at[pl.ds(base, _CHT)], i0a_v)
    pltpu.sync_copy(p1_hbm.at[pl.ds(base, _CHT)], i1a_v)
    pltpu.sync_copy(p0_hbm.at[pl.ds(base + _CHT, _CHT)], i0b_v)
    pltpu.sync_copy(p1_hbm.at[pl.ds(base + _CHT, _CHT)], i1b_v)
    pltpu.sync_copy(g0_hbm.at[pl.ds(base, TPW)], g0_v)
    pltpu.sync_copy(g1_hbm.at[pl.ds(base, TPW)], g1_v)
    ca0 = pltpu.async_copy(ys_hbm.at[i0a_v], r0a_v, s0a)
    ca1 = pltpu.async_copy(ys_hbm.at[i1a_v], r1a_v, s1a)
    cb0 = pltpu.async_copy(ys_hbm.at[i0b_v], r0b_v, s0b)
    cb1 = pltpu.async_copy(ys_hbm.at[i1b_v], r1b_v, s1b)

    def gated_sum(r0_v, r1_v, goff):
        def body(t, _):
            g0s = g0_v[t + goff, :]
            g1s = g1_v[t + goff, :]
            for k in range(D // 16):
                sl = pl.ds(k * 16, 16)
                r0_v[t, sl] = g0s * r0_v[t, sl] + g1s * r1_v[t, sl]
            return 0
        lax.fori_loop(0, _CHT, body, 0)

    ca0.wait()
    ca1.wait()
    gated_sum(r0a_v, r1a_v, 0)
    wa = pltpu.async_copy(r0a_v, out_hbm.at[pl.ds(base, _CHT)], s0a)
    cb0.wait()
    cb1.wait()
    gated_sum(r0b_v, r1b_v, _CHT)
    wa.wait()
    pltpu.sync_copy(r0b_v, out_hbm.at[pl.ds(base + _CHT, _CHT)])


@functools.cache
def _get_combine():
    mesh = plsc.VectorSubcoreMesh(core_axis_name="c", subcore_axis_name="s",
                                  num_cores=NC, num_subcores=NS)
    return pl.kernel(
        _combine_body,
        out_type=jax.ShapeDtypeStruct((N, D), jnp.float32),
        mesh=mesh,
        scratch_types=[
            pltpu.VMEM((_CHT, D), jnp.float32),
            pltpu.VMEM((_CHT, D), jnp.float32),
            pltpu.VMEM((_CHT, D), jnp.float32),
            pltpu.VMEM((_CHT, D), jnp.float32),
            pltpu.VMEM((_CHT,), jnp.int32),
            pltpu.VMEM((_CHT,), jnp.int32),
            pltpu.VMEM((_CHT,), jnp.int32),
            pltpu.VMEM((_CHT,), jnp.int32),
            pltpu.VMEM((TPW, 16), jnp.float32),
            pltpu.VMEM((TPW, 16), jnp.float32),
            pltpu.SemaphoreType.DMA,
            pltpu.SemaphoreType.DMA,
            pltpu.SemaphoreType.DMA,
            pltpu.SemaphoreType.DMA,
        ],
    )


def kernel(x, router_w, router_b, w1, b1, w2, b2):
    xf = x.reshape(N, D)
    g0, g1, p0, p1, te, tot, tid = _router(xf, router_w,
                                           router_b.reshape(1, E))
    p0, p1 = p0.reshape(N), p1.reshape(N)
    rid = _get_dispatch()(tid, p0, p1)
    ys = _gmm(te, tot, rid, xf,
              w1, b1.reshape(E, 1, H),
              w2, b2.reshape(E, 1, D))
    out = _get_combine()(ys, p0, p1, g0, g1)
    return out.reshape(x.shape)


# speedup vs baseline: 1.0282x; 1.0282x over previous
import functools

import jax
import jax.numpy as jnp
from jax import lax
from jax.experimental import pallas as pl
from jax.experimental.pallas import tpu as pltpu
from jax.experimental.pallas import tpu_sc as plsc

N = 2048
D = 768
H = 3072
E = 64
T = 64
NT = 128
PAD = NT * T
CH = 512

NC, NS = 2, 16
NW = NC * NS
TPW = N // NW


def _router_body(xf_ref, rw_ref, rb_ref,
                 g0_ref, g1_ref, p0_ref, p1_ref, te_ref, tot_ref):
    xf = xf_ref[...]
    logits = jnp.dot(xf, rw_ref[...], preferred_element_type=jnp.float32)
    logits = logits + rb_ref[...]
    m = jnp.max(logits, axis=-1, keepdims=True)
    ex = jnp.exp(logits - m)
    p = ex / jnp.sum(ex, axis=-1, keepdims=True)

    lane = lax.broadcasted_iota(jnp.int32, (N, E), 1).astype(jnp.float32)
    m1 = jnp.max(p, axis=-1, keepdims=True)
    i1 = jnp.min(jnp.where(p == m1, lane, float(E + 1)), axis=-1, keepdims=True)
    pmask = jnp.where(lane == i1, -1.0, p)
    m2 = jnp.max(pmask, axis=-1, keepdims=True)
    i2 = jnp.min(jnp.where(pmask == m2, lane, float(E + 1)), axis=-1,
                 keepdims=True)
    g0_ref[...] = jnp.broadcast_to(m1, (N, 16))
    g1_ref[...] = jnp.broadcast_to(m2, (N, 16))

    ea = jnp.concatenate([i1, i2], axis=0)
    lane_c = lax.broadcasted_iota(jnp.int32, (CH, E), 1).astype(jnp.float32)
    r = lax.broadcasted_iota(jnp.int32, (CH, CH), 0).astype(jnp.float32)
    c = lax.broadcasted_iota(jnp.int32, (CH, CH), 1).astype(jnp.float32)
    Ls = (r > c).astype(jnp.float32)

    carry = jnp.zeros((1, E), jnp.float32)
    ranks = []
    for k in range(2 * N // CH):
        Ok = (ea[k * CH:(k + 1) * CH] == lane_c).astype(jnp.float32)
        Ck = jnp.dot(Ls, Ok, preferred_element_type=jnp.float32) + carry
        carry = carry + jnp.sum(Ok, axis=0, keepdims=True)
        ranks.append(jnp.sum(Ck * Ok, axis=-1, keepdims=True))
    rank = jnp.concatenate(ranks, axis=0)
    counts = carry

    tpe = jnp.floor((counts + (T - 1)) * (1.0 / T))
    er = lax.broadcasted_iota(jnp.int32, (E, E), 0).astype(jnp.float32)
    ec = lax.broadcasted_iota(jnp.int32, (E, E), 1).astype(jnp.float32)
    Le = (er > ec).astype(jnp.float32)
    cum_excl = jnp.dot(Le, tpe.reshape(E, 1),
                       preferred_element_type=jnp.float32).reshape(1, E)
    cum_incl = cum_excl + tpe
    total = jnp.sum(tpe)
    padded_off = float(T) * cum_excl

    poa = jnp.zeros((0, 1), jnp.float32)
    poas = []
    for k in range(2 * N // CH):
        Ok = (ea[k * CH:(k + 1) * CH] == lane_c).astype(jnp.float32)
        poas.append(jnp.sum(Ok * padded_off, axis=-1, keepdims=True))
    poa = jnp.concatenate(poas, axis=0)
    pos = (poa + rank).astype(jnp.int32)
    p0_ref[...] = pos[:N]
    p1_ref[...] = pos[N:]

    ti = lax.broadcasted_iota(jnp.int32, (NT, E), 0).astype(jnp.float32)
    te = jnp.sum((cum_incl <= ti).astype(jnp.float32), axis=-1, keepdims=True)
    e_col = lax.broadcasted_iota(jnp.int32, (1, E), 1).astype(jnp.float32)
    last_e = jnp.max(jnp.where(tpe > 0, e_col, -1.0))
    te = jnp.minimum(te, last_e)
    te_ref[...] = te.astype(jnp.int32)
    tot_ref[...] = total.astype(jnp.int32).reshape(1, 1)


_router = pl.pallas_call(
    _router_body,
    out_shape=(
        jax.ShapeDtypeStruct((N, 16), jnp.float32),
        jax.ShapeDtypeStruct((N, 16), jnp.float32),
        jax.ShapeDtypeStruct((N, 1), jnp.int32),
        jax.ShapeDtypeStruct((N, 1), jnp.int32),
        jax.ShapeDtypeStruct((NT, 1), jnp.int32),
        jax.ShapeDtypeStruct((1, 1), jnp.int32),
    ),
)


def _dispatch_body(xf_hbm, p0_hbm, p1_hbm, xs_hbm, rows_v, i0_v, i1_v,
                   s0, s1, sr):
    wid = lax.axis_index("s") * NC + lax.axis_index("c")
    base = wid * TPW
    cr = pltpu.async_copy(xf_hbm.at[pl.ds(base, TPW)], rows_v, sr)
    c0i = pltpu.async_copy(p0_hbm.at[pl.ds(base, TPW)], i0_v, s0)
    c1i = pltpu.async_copy(p1_hbm.at[pl.ds(base, TPW)], i1_v, s1)
    cr.wait()
    c0i.wait()
    c1i.wait()
    c0 = pltpu.async_copy(rows_v, xs_hbm.at[i0_v], s0)
    c1 = pltpu.async_copy(rows_v, xs_hbm.at[i1_v], s1)
    c0.wait()
    c1.wait()


@functools.cache
def _get_dispatch():
    mesh = plsc.VectorSubcoreMesh(core_axis_name="c", subcore_axis_name="s",
                                  num_cores=NC, num_subcores=NS)
    return pl.kernel(
        _dispatch_body,
        out_type=jax.ShapeDtypeStruct((PAD, D), jnp.float32),
        mesh=mesh,
        scratch_types=[
            pltpu.VMEM((TPW, D), jnp.float32),
            pltpu.VMEM((TPW,), jnp.int32),
            pltpu.VMEM((TPW,), jnp.int32),
            pltpu.SemaphoreType.DMA,
            pltpu.SemaphoreType.DMA,
            pltpu.SemaphoreType.DMA,
        ],
    )


_INV_SQRT2 = 0.7071067811865476


_H2 = H // 2


def _gmm_body(te_ref, tot_ref, xs_hbm, w1_hbm, b1_hbm, w2_hbm, b2_hbm, ys_hbm):
    total = tot_ref[0, 0]

    def inner(xs_blk, w1_blk, b1_blk, w2_blk, b2_blk, ys_blk):
        h = jnp.dot(xs_blk[...], w1_blk[0],
                    preferred_element_type=jnp.float32) + b1_blk[0]
        h = 0.5 * h * (1.0 + lax.erf(h * _INV_SQRT2))
        yt = jnp.dot(h, w2_blk[0], preferred_element_type=jnp.float32)
        ys_blk[...] = yt + b2_blk[0]

    wbuf = pl.Buffered(buffer_count=3, use_lookahead=True)
    pipeline = pltpu.emit_pipeline(
        inner,
        grid=(total,),
        in_specs=[
            pl.BlockSpec((T, D), lambda i: (i, 0)),
            pl.BlockSpec((1, D, H), lambda i: (te_ref[i, 0], 0, 0),
                         pipeline_mode=wbuf),
            pl.BlockSpec((1, 1, H), lambda i: (te_ref[i, 0], 0, 0)),
            pl.BlockSpec((1, H, D), lambda i: (te_ref[i, 0], 0, 0),
                         pipeline_mode=wbuf),
            pl.BlockSpec((1, 1, D), lambda i: (te_ref[i, 0], 0, 0)),
        ],
        out_specs=[pl.BlockSpec((T, D), lambda i: (i, 0))],
    )
    pipeline(xs_hbm, w1_hbm, b1_hbm, w2_hbm, b2_hbm, ys_hbm)


_gmm = pl.pallas_call(
    _gmm_body,
    in_specs=[
        pl.BlockSpec(memory_space=pltpu.SMEM),
        pl.BlockSpec(memory_space=pltpu.SMEM),
        pl.BlockSpec(memory_space=pl.ANY),
        pl.BlockSpec(memory_space=pl.ANY),
        pl.BlockSpec(memory_space=pl.ANY),
        pl.BlockSpec(memory_space=pl.ANY),
        pl.BlockSpec(memory_space=pl.ANY),
    ],
    out_specs=pl.BlockSpec(memory_space=pl.ANY),
    out_shape=jax.ShapeDtypeStruct((PAD, D), jnp.float32),
)


_CHT = 32


def _combine_body(ys_hbm, p0_hbm, p1_hbm, g0_hbm, g1_hbm, out_hbm,
                  r0a_v, r1a_v, r0b_v, r1b_v,
                  i0a_v, i1a_v, i0b_v, i1b_v, g0_v, g1_v,
                  s0a, s1a, s0b, s1b):
    wid = lax.axis_index("s") * NC + lax.axis_index("c")
    base = wid * TPW

    pltpu.sync_copy(p0_hbm.at[pl.ds(base, _CHT)], i0a_v)
    pltpu.sync_copy(p1_hbm.at[pl.ds(base, _CHT)], i1a_v)
    pltpu.sync_copy(p0_hbm.at[pl.ds(base + _CHT, _CHT)], i0b_v)
    pltpu.sync_copy(p1_hbm.at[pl.ds(base + _CHT, _CHT)], i1b_v)
    pltpu.sync_copy(g0_hbm.at[pl.ds(base, TPW)], g0_v)
    pltpu.sync_copy(g1_hbm.at[pl.ds(base, TPW)], g1_v)
    ca0 = pltpu.async_copy(ys_hbm.at[i0a_v], r0a_v, s0a)
    ca1 = pltpu.async_copy(ys_hbm.at[i1a_v], r1a_v, s1a)
    cb0 = pltpu.async_copy(ys_hbm.at[i0b_v], r0b_v, s0b)
    cb1 = pltpu.async_copy(ys_hbm.at[i1b_v], r1b_v, s1b)

    def gated_sum(r0_v, r1_v, goff):
        def body(t, _):
            g0s = g0_v[t + goff, :]
            g1s = g1_v[t + goff, :]
            for k in range(D // 16):
                sl = pl.ds(k * 16, 16)
                r0_v[t, sl] = g0s * r0_v[t, sl] + g1s * r1_v[t, sl]
            return 0
        lax.fori_loop(0, _CHT, body, 0)

    ca0.wait()
    ca1.wait()
    gated_sum(r0a_v, r1a_v, 0)
    wa = pltpu.async_copy(r0a_v, out_hbm.at[pl.ds(base, _CHT)], s0a)
    cb0.wait()
    cb1.wait()
    gated_sum(r0b_v, r1b_v, _CHT)
    wa.wait()
    pltpu.sync_copy(r0b_v, out_hbm.at[pl.ds(base + _CHT, _CHT)])


@functools.cache
def _get_combine():
    mesh = plsc.VectorSubcoreMesh(core_axis_name="c", subcore_axis_name="s",
                                  num_cores=NC, num_subcores=NS)
    return pl.kernel(
        _combine_body,
        out_type=jax.ShapeDtypeStruct((N, D), jnp.float32),
        mesh=mesh,
        scratch_types=[
            pltpu.VMEM((_CHT, D), jnp.float32),
            pltpu.VMEM((_CHT, D), jnp.float32),
            pltpu.VMEM((_CHT, D), jnp.float32),
            pltpu.VMEM((_CHT, D), jnp.float32),
            pltpu.VMEM((_CHT,), jnp.int32),
            pltpu.VMEM((_CHT,), jnp.int32),
            pltpu.VMEM((_CHT,), jnp.int32),
            pltpu.VMEM((_CHT,), jnp.int32),
            pltpu.VMEM((TPW, 16), jnp.float32),
            pltpu.VMEM((TPW, 16), jnp.float32),
            pltpu.SemaphoreType.DMA,
            pltpu.SemaphoreType.DMA,
            pltpu.SemaphoreType.DMA,
            pltpu.SemaphoreType.DMA,
        ],
    )


def kernel(x, router_w, router_b, w1, b1, w2, b2):
    xf = x.reshape(N, D)
    g0, g1, p0, p1, te, tot = _router(xf, router_w, router_b.reshape(1, E))
    p0, p1 = p0.reshape(N), p1.reshape(N)
    xs = _get_dispatch()(xf, p0, p1)
    ys = _gmm(te, tot, xs,
              w1, b1.reshape(E, 1, H),
              w2, b2.reshape(E, 1, D))
    out = _get_combine()(ys, p0, p1, g0, g1)
    return out.reshape(x.shape)


# speedup vs baseline: 1.0354x; 1.0070x over previous
import functools

import jax
import jax.numpy as jnp
from jax import lax
from jax.experimental import pallas as pl
from jax.experimental.pallas import tpu as pltpu
from jax.experimental.pallas import tpu_sc as plsc

N = 2048
D = 768
H = 3072
E = 64
T = 64
NT = 128
PAD = NT * T
CH = 512

NC, NS = 2, 16
NW = NC * NS
TPW = N // NW


def _router_body(xf_ref, rw_ref, rb_ref,
                 g0_ref, g1_ref, p0_ref, p1_ref, te_ref, tot_ref):
    xf = xf_ref[...]
    logits = jnp.dot(xf, rw_ref[...], preferred_element_type=jnp.float32)
    logits = logits + rb_ref[...]
    m = jnp.max(logits, axis=-1, keepdims=True)
    ex = jnp.exp(logits - m)
    p = ex / jnp.sum(ex, axis=-1, keepdims=True)

    lane = lax.broadcasted_iota(jnp.int32, (N, E), 1).astype(jnp.float32)
    m1 = jnp.max(p, axis=-1, keepdims=True)
    i1 = jnp.min(jnp.where(p == m1, lane, float(E + 1)), axis=-1, keepdims=True)
    pmask = jnp.where(lane == i1, -1.0, p)
    m2 = jnp.max(pmask, axis=-1, keepdims=True)
    i2 = jnp.min(jnp.where(pmask == m2, lane, float(E + 1)), axis=-1,
                 keepdims=True)
    g0_ref[...] = jnp.broadcast_to(m1, (N, 16))
    g1_ref[...] = jnp.broadcast_to(m2, (N, 16))

    ea = jnp.concatenate([i1, i2], axis=0)
    lane_c = lax.broadcasted_iota(jnp.int32, (CH, E), 1).astype(jnp.float32)
    r = lax.broadcasted_iota(jnp.int32, (CH, CH), 0).astype(jnp.float32)
    c = lax.broadcasted_iota(jnp.int32, (CH, CH), 1).astype(jnp.float32)
    Ls = (r > c).astype(jnp.float32)

    carry = jnp.zeros((1, E), jnp.float32)
    ranks = []
    for k in range(2 * N // CH):
        Ok = (ea[k * CH:(k + 1) * CH] == lane_c).astype(jnp.float32)
        Ck = jnp.dot(Ls, Ok, preferred_element_type=jnp.float32) + carry
        carry = carry + jnp.sum(Ok, axis=0, keepdims=True)
        ranks.append(jnp.sum(Ck * Ok, axis=-1, keepdims=True))
    rank = jnp.concatenate(ranks, axis=0)
    counts = carry

    tpe = jnp.floor((counts + (T - 1)) * (1.0 / T))
    er = lax.broadcasted_iota(jnp.int32, (E, E), 0).astype(jnp.float32)
    ec = lax.broadcasted_iota(jnp.int32, (E, E), 1).astype(jnp.float32)
    Le = (er > ec).astype(jnp.float32)
    cum_excl = jnp.dot(Le, tpe.reshape(E, 1),
                       preferred_element_type=jnp.float32).reshape(1, E)
    cum_incl = cum_excl + tpe
    total = jnp.sum(tpe)
    padded_off = float(T) * cum_excl

    poas = []
    for k in range(2 * N // CH):
        Ok = (ea[k * CH:(k + 1) * CH] == lane_c).astype(jnp.float32)
        poas.append(jnp.dot(Ok, padded_off.reshape(E, 1),
                            preferred_element_type=jnp.float32))
    poa = jnp.concatenate(poas, axis=0)
    pos = (poa + rank).astype(jnp.int32)
    p0_ref[...] = pos[:N]
    p1_ref[...] = pos[N:]

    ti = lax.broadcasted_iota(jnp.int32, (NT, E), 0).astype(jnp.float32)
    te = jnp.sum((cum_incl <= ti).astype(jnp.float32), axis=-1, keepdims=True)
    e_col = lax.broadcasted_iota(jnp.int32, (1, E), 1).astype(jnp.float32)
    last_e = jnp.max(jnp.where(tpe > 0, e_col, -1.0))
    te = jnp.minimum(te, last_e)
    te_ref[...] = te.astype(jnp.int32)
    tot_ref[...] = total.astype(jnp.int32).reshape(1, 1)


_router = pl.pallas_call(
    _router_body,
    out_shape=(
        jax.ShapeDtypeStruct((N, 16), jnp.float32),
        jax.ShapeDtypeStruct((N, 16), jnp.float32),
        jax.ShapeDtypeStruct((N, 1), jnp.int32),
        jax.ShapeDtypeStruct((N, 1), jnp.int32),
        jax.ShapeDtypeStruct((NT, 1), jnp.int32),
        jax.ShapeDtypeStruct((1, 1), jnp.int32),
    ),
)


def _dispatch_body(xf_hbm, p0_hbm, p1_hbm, xs_hbm, rows_v, i0_v, i1_v,
                   s0, s1, sr):
    wid = lax.axis_index("s") * NC + lax.axis_index("c")
    base = wid * TPW
    cr = pltpu.async_copy(xf_hbm.at[pl.ds(base, TPW)], rows_v, sr)
    c0i = pltpu.async_copy(p0_hbm.at[pl.ds(base, TPW)], i0_v, s0)
    c1i = pltpu.async_copy(p1_hbm.at[pl.ds(base, TPW)], i1_v, s1)
    cr.wait()
    c0i.wait()
    c1i.wait()
    c0 = pltpu.async_copy(rows_v, xs_hbm.at[i0_v], s0)
    c1 = pltpu.async_copy(rows_v, xs_hbm.at[i1_v], s1)
    c0.wait()
    c1.wait()


@functools.cache
def _get_dispatch():
    mesh = plsc.VectorSubcoreMesh(core_axis_name="c", subcore_axis_name="s",
                                  num_cores=NC, num_subcores=NS)
    return pl.kernel(
        _dispatch_body,
        out_type=jax.ShapeDtypeStruct((PAD, D), jnp.float32),
        mesh=mesh,
        scratch_types=[
            pltpu.VMEM((TPW, D), jnp.float32),
            pltpu.VMEM((TPW,), jnp.int32),
            pltpu.VMEM((TPW,), jnp.int32),
            pltpu.SemaphoreType.DMA,
            pltpu.SemaphoreType.DMA,
            pltpu.SemaphoreType.DMA,
        ],
    )


_INV_SQRT2 = 0.7071067811865476


_H2 = H // 2


def _gmm_body(te_ref, tot_ref, xs_hbm, w1_hbm, b1_hbm, w2_hbm, b2_hbm, ys_hbm):
    total = tot_ref[0, 0]

    def inner(xs_blk, w1_blk, b1_blk, w2_blk, b2_blk, ys_blk):
        h = jnp.dot(xs_blk[...], w1_blk[0],
                    preferred_element_type=jnp.float32) + b1_blk[0]
        h = 0.5 * h * (1.0 + lax.erf(h * _INV_SQRT2))
        yt = jnp.dot(h, w2_blk[0], preferred_element_type=jnp.float32)
        ys_blk[...] = yt + b2_blk[0]

    wbuf = pl.Buffered(buffer_count=3, use_lookahead=True)
    pipeline = pltpu.emit_pipeline(
        inner,
        grid=(total,),
        in_specs=[
            pl.BlockSpec((T, D), lambda i: (i, 0)),
            pl.BlockSpec((1, D, H), lambda i: (te_ref[i, 0], 0, 0),
                         pipeline_mode=wbuf),
            pl.BlockSpec((1, 1, H), lambda i: (te_ref[i, 0], 0, 0)),
            pl.BlockSpec((1, H, D), lambda i: (te_ref[i, 0], 0, 0),
                         pipeline_mode=wbuf),
            pl.BlockSpec((1, 1, D), lambda i: (te_ref[i, 0], 0, 0)),
        ],
        out_specs=[pl.BlockSpec((T, D), lambda i: (i, 0))],
    )
    pipeline(xs_hbm, w1_hbm, b1_hbm, w2_hbm, b2_hbm, ys_hbm)


_gmm = pl.pallas_call(
    _gmm_body,
    in_specs=[
        pl.BlockSpec(memory_space=pltpu.SMEM),
        pl.BlockSpec(memory_space=pltpu.SMEM),
        pl.BlockSpec(memory_space=pl.ANY),
        pl.BlockSpec(memory_space=pl.ANY),
        pl.BlockSpec(memory_space=pl.ANY),
        pl.BlockSpec(memory_space=pl.ANY),
        pl.BlockSpec(memory_space=pl.ANY),
    ],
    out_specs=pl.BlockSpec(memory_space=pl.ANY),
    out_shape=jax.ShapeDtypeStruct((PAD, D), jnp.float32),
)


_CHT = 32


def _combine_body(ys_hbm, p0_hbm, p1_hbm, g0_hbm, g1_hbm, out_hbm,
                  r0a_v, r1a_v, r0b_v, r1b_v,
                  i0a_v, i1a_v, i0b_v, i1b_v, g0_v, g1_v,
                  s0a, s1a, s0b, s1b):
    wid = lax.axis_index("s") * NC + lax.axis_index("c")
    base = wid * TPW

    pltpu.sync_copy(p0_hbm.at[pl.ds(base, _CHT)], i0a_v)
    pltpu.sync_copy(p1_hbm.at[pl.ds(base, _CHT)], i1a_v)
    pltpu.sync_copy(p0_hbm.at[pl.ds(base + _CHT, _CHT)], i0b_v)
    pltpu.sync_copy(p1_hbm.at[pl.ds(base + _CHT, _CHT)], i1b_v)
    pltpu.sync_copy(g0_hbm.at[pl.ds(base, TPW)], g0_v)
    pltpu.sync_copy(g1_hbm.at[pl.ds(base, TPW)], g1_v)
    ca0 = pltpu.async_copy(ys_hbm.at[i0a_v], r0a_v, s0a)
    ca1 = pltpu.async_copy(ys_hbm.at[i1a_v], r1a_v, s1a)
    cb0 = pltpu.async_copy(ys_hbm.at[i0b_v], r0b_v, s0b)
    cb1 = pltpu.async_copy(ys_hbm.at[i1b_v], r1b_v, s1b)

    def gated_sum(r0_v, r1_v, goff):
        def body(t, _):
            g0s = g0_v[t + goff, :]
            g1s = g1_v[t + goff, :]
            for k in range(D // 16):
                sl = pl.ds(k * 16, 16)
                r0_v[t, sl] = g0s * r0_v[t, sl] + g1s * r1_v[t, sl]
            return 0
        lax.fori_loop(0, _CHT, body, 0)

    ca0.wait()
    ca1.wait()
    gated_sum(r0a_v, r1a_v, 0)
    wa = pltpu.async_copy(r0a_v, out_hbm.at[pl.ds(base, _CHT)], s0a)
    cb0.wait()
    cb1.wait()
    gated_sum(r0b_v, r1b_v, _CHT)
    wa.wait()
    pltpu.sync_copy(r0b_v, out_hbm.at[pl.ds(base + _CHT, _CHT)])


@functools.cache
def _get_combine():
    mesh = plsc.VectorSubcoreMesh(core_axis_name="c", subcore_axis_name="s",
                                  num_cores=NC, num_subcores=NS)
    return pl.kernel(
        _combine_body,
        out_type=jax.ShapeDtypeStruct((N, D), jnp.float32),
        mesh=mesh,
        scratch_types=[
            pltpu.VMEM((_CHT, D), jnp.float32),
            pltpu.VMEM((_CHT, D), jnp.float32),
            pltpu.VMEM((_CHT, D), jnp.float32),
            pltpu.VMEM((_CHT, D), jnp.float32),
            pltpu.VMEM((_CHT,), jnp.int32),
            pltpu.VMEM((_CHT,), jnp.int32),
            pltpu.VMEM((_CHT,), jnp.int32),
            pltpu.VMEM((_CHT,), jnp.int32),
            pltpu.VMEM((TPW, 16), jnp.float32),
            pltpu.VMEM((TPW, 16), jnp.float32),
            pltpu.SemaphoreType.DMA,
            pltpu.SemaphoreType.DMA,
            pltpu.SemaphoreType.DMA,
            pltpu.SemaphoreType.DMA,
        ],
    )


def kernel(x, router_w, router_b, w1, b1, w2, b2):
    xf = x.reshape(N, D)
    g0, g1, p0, p1, te, tot = _router(xf, router_w, router_b.reshape(1, E))
    p0, p1 = p0.reshape(N), p1.reshape(N)
    xs = _get_dispatch()(xf, p0, p1)
    ys = _gmm(te, tot, xs,
              w1, b1.reshape(E, 1, H),
              w2, b2.reshape(E, 1, D))
    out = _get_combine()(ys, p0, p1, g0, g1)
    return out.reshape(x.shape)
